# baseline (speedup 1.0000x reference)
import functools

import jax
import jax.numpy as jnp
from jax import lax
from jax.experimental import pallas as pl
from jax.experimental.pallas import tpu as pltpu
from jax.experimental.pallas import tpu_sc as plsc

_NUM_TILES = 32
_SUB = 128
_FEAT = 128


def _body(x_hbm, labels_hbm, centers_hbm, out_hbm, lbl_v, x_v, c_v, p_v, res_v, sem):
    wid = lax.axis_index("s") * 2 + lax.axis_index("c")
    rows_per_tile = 4 * _SUB
    base = wid * rows_per_tile

    total = jnp.zeros((16,), jnp.float32)
    for s in range(4):
        off = base + s * _SUB
        pltpu.sync_copy(labels_hbm.at[pl.ds(off, _SUB)], lbl_v)
        pltpu.async_copy(centers_hbm.at[lbl_v], c_v, sem).wait()
        pltpu.sync_copy(x_hbm.at[pl.ds(off, _SUB)], x_v)

        def rbody(r, carry):
            p = jnp.zeros((16,), jnp.float32)
            for j in range(_FEAT // 16):
                xv = x_v[r, pl.ds(j * 16, 16)]
                cv = c_v[r, pl.ds(j * 16, 16)]
                d = xv - cv
                p = p + d * d
            p_v[pl.ds(r * 16, 16)] = p
            return carry

        lax.fori_loop(0, _SUB, rbody, 0)

        lanes16 = jnp.arange(16, dtype=jnp.int32) * 16

        def gbody(g, tot):
            acc = jnp.zeros((16,), jnp.float32)
            gb = g * 256
            for l in range(16):
                idx = lanes16 + (gb + l)
                acc = acc + plsc.load_gather(p_v, [idx])
            return tot + jnp.clip(acc, 1e-12, 1e12)

        total = lax.fori_loop(0, _SUB // 16, gbody, total)

    res_v[...] = total
    pltpu.sync_copy(res_v, out_hbm.at[wid])


@functools.partial(jax.jit, static_argnames=())
def kernel(x, labels, centers):
    batch = x.shape[0]
    labels = labels.astype(jnp.int32)

    k = pl.kernel(
        _body,
        out_type=jax.ShapeDtypeStruct((_NUM_TILES, 16), jnp.float32),
        mesh=plsc.VectorSubcoreMesh(core_axis_name="c", subcore_axis_name="s"),
        compiler_params=pltpu.CompilerParams(needs_layout_passes=False),
        scratch_types=[
            pltpu.VMEM((_SUB,), jnp.int32),
            pltpu.VMEM((_SUB, _FEAT), jnp.float32),
            pltpu.VMEM((_SUB, _FEAT), jnp.float32),
            pltpu.VMEM((_SUB * 16,), jnp.float32),
            pltpu.VMEM((16,), jnp.float32),
            pltpu.SemaphoreType.DMA,
        ],
    )
    partials = k(x, labels, centers)
    return jnp.sum(partials) / batch
